# baseline (speedup 1.0000x reference)
import jax
import jax.numpy as jnp
from jax.experimental import pallas as pl
from jax.experimental.pallas import tpu as pltpu

C = 256


def _gru_body(xt_ref, wihT_ref, whhT_ref, bih_ref, bhh_ref, h_ref, wnew_ref):
    xt = xt_ref[...]
    h = h_ref[...]
    gi = jnp.dot(xt, wihT_ref[...], preferred_element_type=jnp.float32) + bih_ref[...]
    gh = jnp.dot(h, whhT_ref[...], preferred_element_type=jnp.float32) + bhh_ref[...]
    i_r, i_z, i_n = gi[:, :C], gi[:, C:2 * C], gi[:, 2 * C:]
    h_r, h_z, h_n = gh[:, :C], gh[:, C:2 * C], gh[:, 2 * C:]
    r = jax.nn.sigmoid(i_r + h_r)
    z = jax.nn.sigmoid(i_z + h_z)
    n_gate = jnp.tanh(i_n + r * h_n)
    wnew_ref[...] = (1.0 - z) * n_gate + z * h


def _gru_evolve(x_tilde, w_ihT, w_hhT, b_ih, b_hh, h0):
    return pl.pallas_call(
        _gru_body,
        out_shape=jax.ShapeDtypeStruct((C, C), jnp.float32),
    )(x_tilde, w_ihT, w_hhT, b_ih.reshape(1, 3 * C), b_hh.reshape(1, 3 * C), h0)


def kernel(X, edge_index, edge_weight, p_attn, W_ih, W_hh, b_ih, b_hh, gcn_weight):
    n = X.shape[0]
    score = (X @ p_attn) / jnp.linalg.norm(p_attn)
    vals, perm = jax.lax.top_k(score, C)
    X_tilde = X[perm] * jnp.tanh(vals)[:, None]
    W_new = _gru_evolve(X_tilde, W_ih.T, W_hh.T, b_ih, b_hh, gcn_weight)
    loop = jnp.arange(n, dtype=edge_index.dtype)
    src = jnp.concatenate([edge_index[0], loop])
    dst = jnp.concatenate([edge_index[1], loop])
    ew = jnp.concatenate([edge_weight, jnp.ones((n,), dtype=X.dtype)])
    deg = jax.ops.segment_sum(ew, dst, num_segments=n)
    dinv = jnp.where(deg > 0, jax.lax.rsqrt(jnp.maximum(deg, 1e-12)), 0.0)
    norm = dinv[src] * ew * dinv[dst]
    H = X @ W_new
    out = jax.ops.segment_sum(H[src] * norm[:, None], dst, num_segments=n)
    return out


# speedup vs baseline: 9.9044x; 9.9044x over previous
import functools

import jax
import jax.numpy as jnp
from jax import lax
from jax.experimental import pallas as pl
from jax.experimental.pallas import tpu as pltpu
from jax.experimental.pallas import tpu_sc as plsc

N = 10000
E = 160000
C = 256

NTILES = 16
HALF = 128
NPAD = 10240
ROWS_PER_TILE = NPAD // NTILES
E_PAD = 172032
EDGES_PER_TILE = E_PAD // NTILES
BLK = 128
HALFE = EDGES_PER_TILE // 2
NBLK_H = HALFE // BLK


def _sc_body(xr, srcp, dstp, ewp, out,
             src_all, dst_all, ew_all, dinv_v,
             idx2_buf, dstblk_buf, ewblk_buf, norm_buf, rows_buf,
             deg_s, g_s, sem):
    cid = lax.axis_index("c")
    sid = lax.axis_index("s")
    ebase = sid * EDGES_PER_TILE
    rbase = sid * ROWS_PER_TILE

    def _zrow(r, _):
        for g in range(8):
            rows_buf[r, pl.ds(g * 16, 16)] = jnp.zeros((16,), jnp.float32)
        return 0
    lax.fori_loop(0, BLK, _zrow, 0)
    for g in range(8):
        ewblk_buf[pl.ds(g * 16, 16)] = jnp.zeros((16,), jnp.float32)
    for k in range(ROWS_PER_TILE // BLK):
        pltpu.sync_copy(rows_buf, g_s.at[pl.ds(rbase + k * BLK, BLK), :])
        pltpu.sync_copy(ewblk_buf, deg_s.at[pl.ds(rbase + k * BLK, BLK)])

    plsc.subcore_barrier()

    def _deg_blk(b, _):
        off = b * BLK
        for j in range(8):
            d16 = dst_all[pl.ds(off + j * 16, 16)]
            e16 = ew_all[pl.ds(off + j * 16, 16)]
            dstblk_buf[pl.ds(j * 16, 16)] = d16
            ewblk_buf[pl.ds(j * 16, 16)] = e16
        pltpu.sync_copy(ewblk_buf, deg_s.at[dstblk_buf], add=True)
        return 0
    for h in range(2):
        pltpu.sync_copy(dstp.at[pl.ds(ebase + h * HALFE, HALFE)], dst_all)
        pltpu.sync_copy(ewp.at[pl.ds(ebase + h * HALFE, HALFE)], ew_all)
        lax.fori_loop(0, NBLK_H, _deg_blk, 0)
    plsc.subcore_barrier()

    pltpu.sync_copy(deg_s, dinv_v)

    def _newton(i, _):
        d = dinv_v[pl.ds(i * 16, 16)]
        bits = lax.bitcast_convert_type(d, jnp.int32)
        y = lax.bitcast_convert_type(jnp.int32(0x5F3759DF) - (bits >> 1), jnp.float32)
        for _ in range(3):
            y = y * (1.5 - 0.5 * d * y * y)
        dinv_v[pl.ds(i * 16, 16)] = y
        return 0
    lax.fori_loop(0, NPAD // 16, _newton, 0)

    def _main_blk(b, _):
        off = b * BLK
        for j in range(8):
            s16 = src_all[pl.ds(off + j * 16, 16)]
            d16 = dst_all[pl.ds(off + j * 16, 16)]
            e16 = ew_all[pl.ds(off + j * 16, 16)]
            idx2_buf[pl.ds(j * 16, 16)] = s16 * 2 + cid
            dstblk_buf[pl.ds(j * 16, 16)] = d16
            a16 = plsc.load_gather(dinv_v, [s16])
            b16 = plsc.load_gather(dinv_v, [d16])
            norm_buf[pl.ds(j * 16, 16)] = a16 * e16 * b16
        pltpu.async_copy(xr.at[idx2_buf], rows_buf, sem).wait()

        def _scale(q, _):
            nv16 = norm_buf[pl.ds(q * 16, 16)]
            for l in range(16):
                r = q * 16 + l
                nv = nv16[l]
                for g in range(8):
                    rows_buf[r, pl.ds(g * 16, 16)] = rows_buf[r, pl.ds(g * 16, 16)] * nv
            return 0
        lax.fori_loop(0, BLK // 16, _scale, 0)
        pltpu.sync_copy(rows_buf, g_s.at[dstblk_buf], add=True)
        return 0
    for h in range(2):
        pltpu.sync_copy(srcp.at[pl.ds(ebase + h * HALFE, HALFE)], src_all)
        pltpu.sync_copy(dstp.at[pl.ds(ebase + h * HALFE, HALFE)], dst_all)
        pltpu.sync_copy(ewp.at[pl.ds(ebase + h * HALFE, HALFE)], ew_all)
        lax.fori_loop(0, NBLK_H, _main_blk, 0)
    plsc.subcore_barrier()

    pltpu.sync_copy(g_s.at[pl.ds(rbase, ROWS_PER_TILE), :],
                    out.at[cid, pl.ds(rbase, ROWS_PER_TILE), :])


def _sc_aggregate(xr, src_p, dst_p, ew_p):
    mesh = plsc.VectorSubcoreMesh(core_axis_name="c", subcore_axis_name="s")
    k = functools.partial(
        pl.kernel,
        mesh=mesh,
        compiler_params=pltpu.CompilerParams(needs_layout_passes=False),
        out_type=jax.ShapeDtypeStruct((2, NPAD, HALF), jnp.float32),
        scratch_types=[
            pltpu.VMEM((HALFE,), jnp.int32),
            pltpu.VMEM((HALFE,), jnp.int32),
            pltpu.VMEM((HALFE,), jnp.float32),
            pltpu.VMEM((NPAD,), jnp.float32),
            pltpu.VMEM((BLK,), jnp.int32),
            pltpu.VMEM((BLK,), jnp.int32),
            pltpu.VMEM((BLK,), jnp.float32),
            pltpu.VMEM((BLK,), jnp.float32),
            pltpu.VMEM((BLK, HALF), jnp.float32),
            pltpu.VMEM_SHARED((NPAD,), jnp.float32),
            pltpu.VMEM_SHARED((NPAD, HALF), jnp.float32),
            pltpu.SemaphoreType.DMA,
        ],
    )(_sc_body)
    return k(xr, src_p, dst_p, ew_p)


def _gru_body(xt_ref, wihT_ref, whhT_ref, bih_ref, bhh_ref, h_ref, wnew_ref):
    xt = xt_ref[...]
    h = h_ref[...]
    gi = jnp.dot(xt, wihT_ref[...], preferred_element_type=jnp.float32) + bih_ref[...]
    gh = jnp.dot(h, whhT_ref[...], preferred_element_type=jnp.float32) + bhh_ref[...]
    i_r, i_z, i_n = gi[:, :C], gi[:, C:2 * C], gi[:, 2 * C:]
    h_r, h_z, h_n = gh[:, :C], gh[:, C:2 * C], gh[:, 2 * C:]
    r = jax.nn.sigmoid(i_r + h_r)
    z = jax.nn.sigmoid(i_z + h_z)
    n_gate = jnp.tanh(i_n + r * h_n)
    wnew_ref[...] = (1.0 - z) * n_gate + z * h


def _gru_evolve(x_tilde, w_ihT, w_hhT, b_ih, b_hh, h0):
    return pl.pallas_call(
        _gru_body,
        out_shape=jax.ShapeDtypeStruct((C, C), jnp.float32),
    )(x_tilde, w_ihT, w_hhT, b_ih.reshape(1, 3 * C), b_hh.reshape(1, 3 * C), h0)


def _matmul_body(g_ref, w_ref, o_ref):
    o_ref[...] = jnp.dot(g_ref[...], w_ref[...], preferred_element_type=jnp.float32)


def _apply_weight(g, w_new):
    return pl.pallas_call(
        _matmul_body,
        grid=(10,),
        in_specs=[
            pl.BlockSpec((1000, C), lambda i: (i, 0)),
            pl.BlockSpec((C, C), lambda i: (0, 0)),
        ],
        out_specs=pl.BlockSpec((1000, C), lambda i: (i, 0)),
        out_shape=jax.ShapeDtypeStruct((N, C), jnp.float32),
    )(g, w_new)


def kernel(X, edge_index, edge_weight, p_attn, W_ih, W_hh, b_ih, b_hh, gcn_weight):
    loop = jnp.arange(N, dtype=jnp.int32)
    pad = E_PAD - E - N
    src_p = jnp.concatenate([edge_index[0], loop, jnp.zeros((pad,), jnp.int32)])
    dst_p = jnp.concatenate([edge_index[1], loop, jnp.full((pad,), N, jnp.int32)])
    ew_p = jnp.concatenate([edge_weight, jnp.ones((N,), jnp.float32),
                            jnp.zeros((pad,), jnp.float32)])
    xr = X.reshape(2 * N, HALF)
    g2 = _sc_aggregate(xr, src_p, dst_p, ew_p)
    G = jnp.concatenate([g2[0, :N, :], g2[1, :N, :]], axis=1)

    score = (X @ p_attn) / jnp.linalg.norm(p_attn)
    vals, perm = jax.lax.top_k(score, C)
    X_tilde = X[perm] * jnp.tanh(vals)[:, None]
    W_new = _gru_evolve(X_tilde, W_ih.T, W_hh.T, b_ih, b_hh, gcn_weight)

    return _apply_weight(G, W_new)


# speedup vs baseline: 11.8752x; 1.1990x over previous
import functools

import jax
import jax.numpy as jnp
from jax import lax
from jax.experimental import pallas as pl
from jax.experimental.pallas import tpu as pltpu
from jax.experimental.pallas import tpu_sc as plsc

N = 10000
E = 160000
C = 256

NTILES = 16
HALF = 128
NPAD = 10240
NDINV = 10016
ROWS_PER_TILE = NPAD // NTILES
E_PAD = 172032
EDGES_PER_TILE = E_PAD // NTILES
BLK = 96
NBLK = EDGES_PER_TILE // BLK
SBLK = 28
NSTAGE = NBLK // SBLK
STAGEE = SBLK * BLK


def _sc_body(xr, srcp, dstp, ewp, out,
             src_s, dst_s, ew_s, dinv_v,
             idx2_a, dstb_a, norm_a, rows_a, sem_a,
             idx2_b, dstb_b, norm_b, rows_b, sem_b,
             deg_s, g_s):
    cid = lax.axis_index("c")
    sid = lax.axis_index("s")
    ebase = sid * EDGES_PER_TILE
    rbase = sid * ROWS_PER_TILE

    def _zrow(r, _):
        for g in range(8):
            rows_a[r, pl.ds(g * 16, 16)] = jnp.zeros((16,), jnp.float32)
        return 0
    lax.fori_loop(0, BLK, _zrow, 0)
    for g in range(6):
        norm_a[pl.ds(g * 16, 16)] = jnp.zeros((16,), jnp.float32)
    for k in range(6):
        pltpu.sync_copy(rows_a, g_s.at[pl.ds(rbase + k * BLK, BLK), :])
        pltpu.sync_copy(norm_a, deg_s.at[pl.ds(rbase + k * BLK, BLK)])
    pltpu.sync_copy(rows_a.at[pl.ds(0, 64), :],
                    g_s.at[pl.ds(rbase + 6 * BLK, 64), :])
    pltpu.sync_copy(norm_a.at[pl.ds(0, 64)],
                    deg_s.at[pl.ds(rbase + 6 * BLK, 64)])
    plsc.subcore_barrier()

    def _deg_blk(b, _):
        off = b * BLK
        for j in range(6):
            dstb_a[pl.ds(j * 16, 16)] = dst_s[pl.ds(off + j * 16, 16)]
            norm_a[pl.ds(j * 16, 16)] = ew_s[pl.ds(off + j * 16, 16)]
        pltpu.sync_copy(norm_a, deg_s.at[dstb_a], add=True)
        return 0
    for s in range(NSTAGE):
        sbase = ebase + s * STAGEE
        pltpu.sync_copy(dstp.at[pl.ds(sbase, STAGEE)], dst_s)
        pltpu.sync_copy(ewp.at[pl.ds(sbase, STAGEE)], ew_s)
        lax.fori_loop(0, SBLK, _deg_blk, 0)
    plsc.subcore_barrier()

    pltpu.sync_copy(deg_s.at[pl.ds(0, NDINV)], dinv_v)

    def _newton(i, _):
        d = dinv_v[pl.ds(i * 16, 16)]
        bits = lax.bitcast_convert_type(d, jnp.int32)
        y = lax.bitcast_convert_type(jnp.int32(0x5F3759DF) - (bits >> 1), jnp.float32)
        for _ in range(3):
            y = y * (1.5 - 0.5 * d * y * y)
        dinv_v[pl.ds(i * 16, 16)] = y
        return 0
    lax.fori_loop(0, NDINV // 16, _newton, 0)

    def _prep(b, idx2, dstb, norm):
        off = b * BLK
        for j in range(6):
            s16 = src_s[pl.ds(off + j * 16, 16)]
            d16 = dst_s[pl.ds(off + j * 16, 16)]
            e16 = ew_s[pl.ds(off + j * 16, 16)]
            idx2[pl.ds(j * 16, 16)] = s16 * 2 + cid
            dstb[pl.ds(j * 16, 16)] = d16
            a16 = plsc.load_gather(dinv_v, [s16])
            b16 = plsc.load_gather(dinv_v, [d16])
            norm[pl.ds(j * 16, 16)] = a16 * e16 * b16

    def _gstart(idx2, rows, sem):
        pltpu.async_copy(xr.at[idx2], rows, sem)

    def _fin(idx2, dstb, norm, rows, sem):
        pltpu.make_async_copy(xr.at[idx2], rows, sem).wait()

        def _scale(q, _):
            nv16 = norm[pl.ds(q * 16, 16)]
            for l in range(16):
                r = q * 16 + l
                nv = nv16[l]
                for g in range(8):
                    rows[r, pl.ds(g * 16, 16)] = rows[r, pl.ds(g * 16, 16)] * nv
            return 0
        lax.fori_loop(0, BLK // 16, _scale, 0)
        pltpu.sync_copy(rows, g_s.at[dstb], add=True)

    seta = (idx2_a, dstb_a, norm_a, rows_a, sem_a)
    setb = (idx2_b, dstb_b, norm_b, rows_b, sem_b)

    def _pair(i, _):
        _prep(2 * i + 1, idx2_b, dstb_b, norm_b)
        _gstart(idx2_b, rows_b, sem_b)
        _fin(*seta)
        _prep(2 * i + 2, idx2_a, dstb_a, norm_a)
        _gstart(idx2_a, rows_a, sem_a)
        _fin(*setb)
        return 0

    for s in range(NSTAGE):
        sbase = ebase + s * STAGEE
        pltpu.sync_copy(srcp.at[pl.ds(sbase, STAGEE)], src_s)
        pltpu.sync_copy(dstp.at[pl.ds(sbase, STAGEE)], dst_s)
        pltpu.sync_copy(ewp.at[pl.ds(sbase, STAGEE)], ew_s)
        _prep(0, idx2_a, dstb_a, norm_a)
        _gstart(idx2_a, rows_a, sem_a)
        lax.fori_loop(0, SBLK // 2 - 1, _pair, 0)
        _prep(SBLK - 1, idx2_b, dstb_b, norm_b)
        _gstart(idx2_b, rows_b, sem_b)
        _fin(*seta)
        _fin(*setb)
    plsc.subcore_barrier()

    pltpu.sync_copy(g_s.at[pl.ds(rbase, ROWS_PER_TILE), :],
                    out.at[cid, pl.ds(rbase, ROWS_PER_TILE), :])


def _sc_aggregate(xr, src_p, dst_p, ew_p):
    mesh = plsc.VectorSubcoreMesh(core_axis_name="c", subcore_axis_name="s")
    k = functools.partial(
        pl.kernel,
        mesh=mesh,
        compiler_params=pltpu.CompilerParams(needs_layout_passes=False),
        out_type=jax.ShapeDtypeStruct((2, NPAD, HALF), jnp.float32),
        scratch_types=[
            pltpu.VMEM((STAGEE,), jnp.int32),
            pltpu.VMEM((STAGEE,), jnp.int32),
            pltpu.VMEM((STAGEE,), jnp.float32),
            pltpu.VMEM((NDINV,), jnp.float32),
            pltpu.VMEM((BLK,), jnp.int32),
            pltpu.VMEM((BLK,), jnp.int32),
            pltpu.VMEM((BLK,), jnp.float32),
            pltpu.VMEM((BLK, HALF), jnp.float32),
            pltpu.SemaphoreType.DMA,
            pltpu.VMEM((BLK,), jnp.int32),
            pltpu.VMEM((BLK,), jnp.int32),
            pltpu.VMEM((BLK,), jnp.float32),
            pltpu.VMEM((BLK, HALF), jnp.float32),
            pltpu.SemaphoreType.DMA,
            pltpu.VMEM_SHARED((NPAD,), jnp.float32),
            pltpu.VMEM_SHARED((NPAD, HALF), jnp.float32),
        ],
    )(_sc_body)
    return k(xr, src_p, dst_p, ew_p)


def _gru_body(xt_ref, wihT_ref, whhT_ref, bih_ref, bhh_ref, h_ref, wnew_ref):
    xt = xt_ref[...]
    h = h_ref[...]
    gi = jnp.dot(xt, wihT_ref[...], preferred_element_type=jnp.float32) + bih_ref[...]
    gh = jnp.dot(h, whhT_ref[...], preferred_element_type=jnp.float32) + bhh_ref[...]
    i_r, i_z, i_n = gi[:, :C], gi[:, C:2 * C], gi[:, 2 * C:]
    h_r, h_z, h_n = gh[:, :C], gh[:, C:2 * C], gh[:, 2 * C:]
    r = jax.nn.sigmoid(i_r + h_r)
    z = jax.nn.sigmoid(i_z + h_z)
    n_gate = jnp.tanh(i_n + r * h_n)
    wnew_ref[...] = (1.0 - z) * n_gate + z * h


def _gru_evolve(x_tilde, w_ihT, w_hhT, b_ih, b_hh, h0):
    return pl.pallas_call(
        _gru_body,
        out_shape=jax.ShapeDtypeStruct((C, C), jnp.float32),
    )(x_tilde, w_ihT, w_hhT, b_ih.reshape(1, 3 * C), b_hh.reshape(1, 3 * C), h0)


def _matmul_body(g_ref, w_ref, o_ref):
    o_ref[...] = jnp.dot(g_ref[...], w_ref[...], preferred_element_type=jnp.float32)


def _apply_weight(g, w_new):
    return pl.pallas_call(
        _matmul_body,
        grid=(10,),
        in_specs=[
            pl.BlockSpec((1000, C), lambda i: (i, 0)),
            pl.BlockSpec((C, C), lambda i: (0, 0)),
        ],
        out_specs=pl.BlockSpec((1000, C), lambda i: (i, 0)),
        out_shape=jax.ShapeDtypeStruct((N, C), jnp.float32),
    )(g, w_new)


def kernel(X, edge_index, edge_weight, p_attn, W_ih, W_hh, b_ih, b_hh, gcn_weight):
    loop = jnp.arange(N, dtype=jnp.int32)
    pad = E_PAD - E - N
    src_p = jnp.concatenate([edge_index[0], loop, jnp.zeros((pad,), jnp.int32)])
    dst_p = jnp.concatenate([edge_index[1], loop, jnp.full((pad,), N, jnp.int32)])
    ew_p = jnp.concatenate([edge_weight, jnp.ones((N,), jnp.float32),
                            jnp.zeros((pad,), jnp.float32)])
    xr = X.reshape(2 * N, HALF)
    g2 = _sc_aggregate(xr, src_p, dst_p, ew_p)
    G = jnp.concatenate([g2[0, :N, :], g2[1, :N, :]], axis=1)

    score = (X @ p_attn) / jnp.linalg.norm(p_attn)
    vals, perm = jax.lax.top_k(score, C)
    X_tilde = X[perm] * jnp.tanh(vals)[:, None]
    W_new = _gru_evolve(X_tilde, W_ih.T, W_hh.T, b_ih, b_hh, gcn_weight)

    return _apply_weight(G, W_new)


# speedup vs baseline: 12.0333x; 1.0133x over previous
import functools

import jax
import jax.numpy as jnp
from jax import lax
from jax.experimental import pallas as pl
from jax.experimental.pallas import tpu as pltpu
from jax.experimental.pallas import tpu_sc as plsc

N = 10000
E = 160000
C = 256

NTILES = 16
HALF = 128
NPAD = 10112
NDINV = 10016
ROWS_PER_TILE = NPAD // NTILES
ZTAIL = 120
E_PAD = 172032
EDGES_PER_TILE = E_PAD // NTILES
BLK = 128
NBLK = EDGES_PER_TILE // BLK
SBLK = 14
NSTAGE = NBLK // SBLK
STAGEE = SBLK * BLK


def _sc_body(xr, srcp, dstp, ewp, out,
             src_s, dst_s, ew_s, dinv_v,
             idx2_a, dstb_a, norm_a, rows_a, sem_a,
             idx2_b, dstb_b, norm_b, rows_b, sem_b,
             deg_s, g_s):
    cid = lax.axis_index("c")
    sid = lax.axis_index("s")
    ebase = sid * EDGES_PER_TILE
    rbase = sid * ROWS_PER_TILE

    def _zrow(r, _):
        for g in range(8):
            rows_a[r, pl.ds(g * 16, 16)] = jnp.zeros((16,), jnp.float32)
        return 0
    lax.fori_loop(0, BLK, _zrow, 0)
    for g in range(8):
        norm_a[pl.ds(g * 16, 16)] = jnp.zeros((16,), jnp.float32)
    for k in range(4):
        pltpu.sync_copy(rows_a, g_s.at[pl.ds(rbase + k * BLK, BLK), :])
        pltpu.sync_copy(norm_a, deg_s.at[pl.ds(rbase + k * BLK, BLK)])
    pltpu.sync_copy(rows_a.at[pl.ds(0, ZTAIL), :],
                    g_s.at[pl.ds(rbase + 4 * BLK, ZTAIL), :])
    pltpu.sync_copy(norm_a.at[pl.ds(0, ZTAIL)],
                    deg_s.at[pl.ds(rbase + 4 * BLK, ZTAIL)])
    plsc.subcore_barrier()

    def _deg_blk(b, _):
        off = b * BLK
        for j in range(8):
            dstb_a[pl.ds(j * 16, 16)] = dst_s[pl.ds(off + j * 16, 16)]
            norm_a[pl.ds(j * 16, 16)] = ew_s[pl.ds(off + j * 16, 16)]
        pltpu.sync_copy(norm_a, deg_s.at[dstb_a], add=True)
        return 0
    def _deg_stage(s, _):
        sbase = ebase + s * STAGEE
        pltpu.sync_copy(dstp.at[pl.ds(sbase, STAGEE)], dst_s)
        pltpu.sync_copy(ewp.at[pl.ds(sbase, STAGEE)], ew_s)
        lax.fori_loop(0, SBLK, _deg_blk, 0)
        return 0
    lax.fori_loop(0, NSTAGE, _deg_stage, 0)
    plsc.subcore_barrier()

    pltpu.sync_copy(deg_s.at[pl.ds(0, NDINV)], dinv_v)

    def _newton(i, _):
        d = dinv_v[pl.ds(i * 16, 16)]
        bits = lax.bitcast_convert_type(d, jnp.int32)
        y = lax.bitcast_convert_type(jnp.int32(0x5F3759DF) - (bits >> 1), jnp.float32)
        for _ in range(3):
            y = y * (1.5 - 0.5 * d * y * y)
        dinv_v[pl.ds(i * 16, 16)] = y
        return 0
    lax.fori_loop(0, NDINV // 16, _newton, 0)

    def _prep(b, idx2, dstb, norm):
        off = b * BLK
        for j in range(8):
            s16 = src_s[pl.ds(off + j * 16, 16)]
            d16 = dst_s[pl.ds(off + j * 16, 16)]
            e16 = ew_s[pl.ds(off + j * 16, 16)]
            idx2[pl.ds(j * 16, 16)] = s16 * 2 + cid
            dstb[pl.ds(j * 16, 16)] = d16
            a16 = plsc.load_gather(dinv_v, [s16])
            b16 = plsc.load_gather(dinv_v, [d16])
            norm[pl.ds(j * 16, 16)] = a16 * e16 * b16

    def _gstart(idx2, rows, sem):
        pltpu.async_copy(xr.at[idx2], rows, sem)

    def _fin(idx2, dstb, norm, rows, sem):
        pltpu.make_async_copy(xr.at[idx2], rows, sem).wait()

        def _scale(q, _):
            nv16 = norm[pl.ds(q * 16, 16)]
            for l in range(16):
                r = q * 16 + l
                nv = nv16[l]
                for g in range(8):
                    rows[r, pl.ds(g * 16, 16)] = rows[r, pl.ds(g * 16, 16)] * nv
            return 0
        lax.fori_loop(0, BLK // 16, _scale, 0)
        pltpu.sync_copy(rows, g_s.at[dstb], add=True)

    seta = (idx2_a, dstb_a, norm_a, rows_a, sem_a)
    setb = (idx2_b, dstb_b, norm_b, rows_b, sem_b)

    def _pair(i, _):
        _prep(2 * i + 1, idx2_b, dstb_b, norm_b)
        _gstart(idx2_b, rows_b, sem_b)
        _fin(*seta)
        _prep(2 * i + 2, idx2_a, dstb_a, norm_a)
        _gstart(idx2_a, rows_a, sem_a)
        _fin(*setb)
        return 0

    def _gather_stage(s, _):
        sbase = ebase + s * STAGEE
        pltpu.sync_copy(srcp.at[pl.ds(sbase, STAGEE)], src_s)
        pltpu.sync_copy(dstp.at[pl.ds(sbase, STAGEE)], dst_s)
        pltpu.sync_copy(ewp.at[pl.ds(sbase, STAGEE)], ew_s)
        _prep(0, idx2_a, dstb_a, norm_a)
        _gstart(idx2_a, rows_a, sem_a)
        lax.fori_loop(0, SBLK // 2 - 1, _pair, 0)
        _prep(SBLK - 1, idx2_b, dstb_b, norm_b)
        _gstart(idx2_b, rows_b, sem_b)
        _fin(*seta)
        _fin(*setb)
        return 0
    lax.fori_loop(0, NSTAGE, _gather_stage, 0)
    plsc.subcore_barrier()

    pltpu.sync_copy(g_s.at[pl.ds(rbase, ROWS_PER_TILE), :],
                    out.at[cid, pl.ds(rbase, ROWS_PER_TILE), :])


def _sc_aggregate(xr, src_p, dst_p, ew_p):
    mesh = plsc.VectorSubcoreMesh(core_axis_name="c", subcore_axis_name="s")
    k = functools.partial(
        pl.kernel,
        mesh=mesh,
        compiler_params=pltpu.CompilerParams(needs_layout_passes=False),
        out_type=jax.ShapeDtypeStruct((2, NPAD, HALF), jnp.float32),
        scratch_types=[
            pltpu.VMEM((STAGEE,), jnp.int32),
            pltpu.VMEM((STAGEE,), jnp.int32),
            pltpu.VMEM((STAGEE,), jnp.float32),
            pltpu.VMEM((NDINV,), jnp.float32),
            pltpu.VMEM((BLK,), jnp.int32),
            pltpu.VMEM((BLK,), jnp.int32),
            pltpu.VMEM((BLK,), jnp.float32),
            pltpu.VMEM((BLK, HALF), jnp.float32),
            pltpu.SemaphoreType.DMA,
            pltpu.VMEM((BLK,), jnp.int32),
            pltpu.VMEM((BLK,), jnp.int32),
            pltpu.VMEM((BLK,), jnp.float32),
            pltpu.VMEM((BLK, HALF), jnp.float32),
            pltpu.SemaphoreType.DMA,
            pltpu.VMEM_SHARED((NPAD,), jnp.float32),
            pltpu.VMEM_SHARED((NPAD, HALF), jnp.float32),
        ],
    )(_sc_body)
    return k(xr, src_p, dst_p, ew_p)


def _gru_body(xt_ref, wihT_ref, whhT_ref, bih_ref, bhh_ref, h_ref, wnew_ref):
    xt = xt_ref[...]
    h = h_ref[...]
    gi = jnp.dot(xt, wihT_ref[...], preferred_element_type=jnp.float32) + bih_ref[...]
    gh = jnp.dot(h, whhT_ref[...], preferred_element_type=jnp.float32) + bhh_ref[...]
    i_r, i_z, i_n = gi[:, :C], gi[:, C:2 * C], gi[:, 2 * C:]
    h_r, h_z, h_n = gh[:, :C], gh[:, C:2 * C], gh[:, 2 * C:]
    r = jax.nn.sigmoid(i_r + h_r)
    z = jax.nn.sigmoid(i_z + h_z)
    n_gate = jnp.tanh(i_n + r * h_n)
    wnew_ref[...] = (1.0 - z) * n_gate + z * h


def _gru_evolve(x_tilde, w_ihT, w_hhT, b_ih, b_hh, h0):
    return pl.pallas_call(
        _gru_body,
        out_shape=jax.ShapeDtypeStruct((C, C), jnp.float32),
    )(x_tilde, w_ihT, w_hhT, b_ih.reshape(1, 3 * C), b_hh.reshape(1, 3 * C), h0)


def _matmul_body(g_ref, w_ref, o_ref):
    o_ref[...] = jnp.dot(g_ref[...], w_ref[...], preferred_element_type=jnp.float32)


def _apply_weight(g, w_new):
    return pl.pallas_call(
        _matmul_body,
        grid=(10,),
        in_specs=[
            pl.BlockSpec((1000, C), lambda i: (i, 0)),
            pl.BlockSpec((C, C), lambda i: (0, 0)),
        ],
        out_specs=pl.BlockSpec((1000, C), lambda i: (i, 0)),
        out_shape=jax.ShapeDtypeStruct((N, C), jnp.float32),
    )(g, w_new)


def kernel(X, edge_index, edge_weight, p_attn, W_ih, W_hh, b_ih, b_hh, gcn_weight):
    loop = jnp.arange(N, dtype=jnp.int32)
    pad = E_PAD - E - N
    src_p = jnp.concatenate([edge_index[0], loop, jnp.zeros((pad,), jnp.int32)])
    dst_p = jnp.concatenate([edge_index[1], loop, jnp.full((pad,), N, jnp.int32)])
    ew_p = jnp.concatenate([edge_weight, jnp.ones((N,), jnp.float32),
                            jnp.zeros((pad,), jnp.float32)])
    xr = X.reshape(2 * N, HALF)
    g2 = _sc_aggregate(xr, src_p, dst_p, ew_p)
    G = jnp.concatenate([g2[0, :N, :], g2[1, :N, :]], axis=1)

    score = (X @ p_attn) / jnp.linalg.norm(p_attn)
    vals, perm = jax.lax.top_k(score, C)
    X_tilde = X[perm] * jnp.tanh(vals)[:, None]
    W_new = _gru_evolve(X_tilde, W_ih.T, W_hh.T, b_ih, b_hh, gcn_weight)

    return _apply_weight(G, W_new)
